# baseline (speedup 1.0000x reference)
import functools

import jax
import jax.numpy as jnp
from jax import lax
from jax.experimental import pallas as pl
from jax.experimental.pallas import tpu as pltpu
from jax.experimental.pallas import tpu_sc as plsc

_G = 4


def _broadcast_kernel(lv_ref, out_ref):
    out_ref[...] = jnp.broadcast_to(lv_ref[...], out_ref.shape)


def _make_sc_gather(B, L, D):
    info = plsc.get_sparse_core_info()
    NC, NS, LN = info.num_cores, info.num_subcores, info.num_lanes
    NW = NC * NS
    bw = B // NW
    nchunk = L // LN
    mesh = plsc.VectorSubcoreMesh(core_axis_name="c", subcore_axis_name="s")

    @functools.partial(
        pl.kernel,
        mesh=mesh,
        out_type=jax.ShapeDtypeStruct((B, D), jnp.float32),
        scratch_types=[
            pltpu.VMEM((bw, L), jnp.int32),
            pltpu.VMEM((bw, LN), jnp.int32),
            pltpu.VMEM((LN,), jnp.int32),
            pltpu.VMEM((LN, D), jnp.float32),
            pltpu.SemaphoreType.DMA,
        ],
    )
    def sc_gather(mask_hbm, vflat_hbm, out_hbm, mask_v, acc_v, idx_v, rows_v, sem):
        wid = lax.axis_index("s") * NC + lax.axis_index("c")
        base = wid * bw
        pltpu.sync_copy(mask_hbm.at[pl.ds(base, bw)], mask_v)

        lanes = lax.iota(jnp.int32, LN)
        for r in range(bw):
            def scan_chunk(c, acc):
                m = mask_v[r, pl.ds(c * LN, LN)]
                pos = c * LN + lanes
                return jnp.maximum(acc, jnp.where(m == 0, pos, -1))

            acc_v[r, :] = lax.fori_loop(
                0, nchunk, scan_chunk, jnp.full((LN,), -1, jnp.int32)
            )

        idxvec = jnp.zeros((LN,), jnp.int32)
        for r in range(bw):
            av = acc_v[r, :]
            h = av[0]
            for i in range(1, LN):
                h = jnp.maximum(h, av[i])
            gidx = (base + r) * L + jnp.maximum(h, 0)
            idxvec = jnp.where(lanes == r, gidx, idxvec)
        idx_v[...] = idxvec

        pltpu.async_copy(vflat_hbm.at[idx_v], rows_v, sem).wait()
        pltpu.sync_copy(rows_v.at[pl.ds(0, bw)], out_hbm.at[pl.ds(base, bw)])

    return sc_gather


def kernel(input_values, input_timestamps, is_target_mask, dummy):
    B, L, D = input_values.shape
    mask_i32 = is_target_mask.astype(jnp.int32)
    vflat = input_values.reshape(B * L, D)

    last_values = _make_sc_gather(B, L, D)(mask_i32, vflat)

    out = pl.pallas_call(
        _broadcast_kernel,
        grid=(B // _G,),
        in_specs=[pl.BlockSpec((_G, 1, D), lambda i: (i, 0, 0))],
        out_specs=pl.BlockSpec((_G, L, D), lambda i: (i, 0, 0)),
        out_shape=jax.ShapeDtypeStruct((B, L, D), jnp.float32),
    )(last_values.reshape(B, 1, D))
    return out


# speedup vs baseline: 1.3040x; 1.3040x over previous
import jax
import jax.numpy as jnp
from jax import lax
from jax.experimental import pallas as pl
from jax.experimental.pallas import tpu as pltpu

_NBUF = 8


def _gather_kernel(mask_ref, vals_ref, out_ref, idx_vmem, idx_smem, sem):
    B, L = mask_ref.shape
    pos = lax.broadcasted_iota(jnp.int32, (B, L), 1)
    m = mask_ref[...].astype(jnp.int32)
    cand = jnp.where(m == 0, pos, -1)
    idx_vmem[...] = jnp.maximum(jnp.max(cand, axis=1), 0)
    cp = pltpu.make_async_copy(idx_vmem, idx_smem, sem)
    cp.start()
    cp.wait()

    def fire(b, _):
        pltpu.make_async_copy(
            vals_ref.at[b, idx_smem[b]], out_ref.at[b, 0], sem
        ).start()
        return 0

    def drain(b, _):
        pltpu.make_async_copy(vals_ref.at[b, 0], out_ref.at[b, 0], sem).wait()
        return 0

    lax.fori_loop(0, B, fire, 0)
    lax.fori_loop(0, B, drain, 0)


def _bcast_kernel(lv_ref, out_ref, rep, sems):
    B = lv_ref.shape[0]
    _, L, D = rep.shape

    def body(b, _):
        j = lax.rem(b, _NBUF)

        @pl.when(b >= _NBUF)
        def _():
            pltpu.make_async_copy(
                rep.at[j], out_ref.at[b - _NBUF], sems.at[j]
            ).wait()

        row = lv_ref[pl.ds(b, 1), 0, :]
        rep[pl.ds(j, 1)] = jnp.broadcast_to(row[None], (1, L, D))
        pltpu.make_async_copy(rep.at[j], out_ref.at[b], sems.at[j]).start()
        return 0

    lax.fori_loop(0, B, body, 0)

    def drain(k, _):
        b = B - _NBUF + k
        pltpu.make_async_copy(
            rep.at[lax.rem(b, _NBUF)], out_ref.at[b], sems.at[lax.rem(b, _NBUF)]
        ).wait()
        return 0

    lax.fori_loop(0, _NBUF, drain, 0)


def kernel(input_values, input_timestamps, is_target_mask, dummy):
    B, L, D = input_values.shape
    mask_i8 = is_target_mask.astype(jnp.int8)

    last_values = pl.pallas_call(
        _gather_kernel,
        in_specs=[
            pl.BlockSpec(memory_space=pltpu.VMEM),
            pl.BlockSpec(memory_space=pl.ANY),
        ],
        out_specs=pl.BlockSpec(memory_space=pltpu.VMEM),
        scratch_shapes=[
            pltpu.VMEM((B,), jnp.int32),
            pltpu.SMEM((B,), jnp.int32),
            pltpu.SemaphoreType.DMA,
        ],
        out_shape=jax.ShapeDtypeStruct((B, 1, D), jnp.float32),
    )(mask_i8, input_values)

    out = pl.pallas_call(
        _bcast_kernel,
        in_specs=[pl.BlockSpec(memory_space=pltpu.VMEM)],
        out_specs=pl.BlockSpec(memory_space=pl.ANY),
        scratch_shapes=[
            pltpu.VMEM((_NBUF, L, D), jnp.float32),
            pltpu.SemaphoreType.DMA((_NBUF,)),
        ],
        out_shape=jax.ShapeDtypeStruct((B, L, D), jnp.float32),
    )(last_values)
    return out


# speedup vs baseline: 1.3460x; 1.0322x over previous
import jax
import jax.numpy as jnp
from jax import lax
from jax.experimental import pallas as pl
from jax.experimental.pallas import tpu as pltpu

_NBUF = 8


def _fused_kernel(mask_ref, vals_ref, out_ref, idx_vmem, idx_smem, lv, rep,
                  isem, gsem, sems):
    B, L = mask_ref.shape
    D = lv.shape[-1]

    pos = lax.broadcasted_iota(jnp.int32, (B, L), 1)
    m = mask_ref[...].astype(jnp.int32)
    cand = jnp.where(m == 0, pos, -1)
    idx_vmem[...] = jnp.maximum(jnp.max(cand, axis=1), 0)
    cp = pltpu.make_async_copy(idx_vmem, idx_smem, isem)
    cp.start()
    cp.wait()

    def fire(b, _):
        pltpu.make_async_copy(
            vals_ref.at[b, idx_smem[b]], lv.at[b, 0], gsem
        ).start()
        return 0

    lax.fori_loop(0, B, fire, 0)

    def body(b, _):
        pltpu.make_async_copy(vals_ref.at[b, 0], lv.at[b, 0], gsem).wait()
        j = lax.rem(b, _NBUF)

        @pl.when(b >= _NBUF)
        def _():
            pltpu.make_async_copy(
                rep.at[j], out_ref.at[b - _NBUF], sems.at[j]
            ).wait()

        row = lv[pl.ds(b, 1), 0, :]
        rep[pl.ds(j, 1)] = jnp.broadcast_to(row[None], (1, L, D))
        pltpu.make_async_copy(rep.at[j], out_ref.at[b], sems.at[j]).start()
        return 0

    lax.fori_loop(0, B, body, 0)

    def drain(k, _):
        b = B - _NBUF + k
        pltpu.make_async_copy(
            rep.at[lax.rem(b, _NBUF)], out_ref.at[b], sems.at[lax.rem(b, _NBUF)]
        ).wait()
        return 0

    lax.fori_loop(0, _NBUF, drain, 0)


def kernel(input_values, input_timestamps, is_target_mask, dummy):
    B, L, D = input_values.shape
    mask_i8 = is_target_mask.view(jnp.int8)

    out = pl.pallas_call(
        _fused_kernel,
        in_specs=[
            pl.BlockSpec(memory_space=pltpu.VMEM),
            pl.BlockSpec(memory_space=pl.ANY),
        ],
        out_specs=pl.BlockSpec(memory_space=pl.ANY),
        scratch_shapes=[
            pltpu.VMEM((B,), jnp.int32),
            pltpu.SMEM((B,), jnp.int32),
            pltpu.VMEM((B, 1, D), jnp.float32),
            pltpu.VMEM((_NBUF, L, D), jnp.float32),
            pltpu.SemaphoreType.DMA,
            pltpu.SemaphoreType.DMA,
            pltpu.SemaphoreType.DMA((_NBUF,)),
        ],
        out_shape=jax.ShapeDtypeStruct((B, L, D), jnp.float32),
    )(mask_i8, input_values)
    return out
